# baseline (speedup 1.0000x reference)
import functools
import math

import jax
import jax.numpy as jnp
from jax import lax
from jax.experimental import pallas as pl
from jax.experimental.pallas import tpu as pltpu
from jax.experimental.pallas import tpu_sc as plsc

EMBED = 64
HALF = 32
NTYPES = 100
TPAD = 128

B, S = 4096, 200
N = B * S


PRO_ROWS = 128
PRO_GRID = (N // 128) // PRO_ROWS


def _tc_prologue(dts_ref, table_ref, tw_ref, tb_ref, ew_ref, eb_ref,
                 dt2_ref, dtab_ref, vrow_ref):
    ftab = (table_ref[...] * math.sqrt(EMBED // 2)) @ ew_ref[HALF:, :]
    c = tb_ref[...] @ ew_ref[:HALF, :] + eb_ref[...]
    v = tw_ref[...] @ ew_ref[:HALF, :]
    dtab_ref[...] = 2.0 * (ftab + c)
    vrow_ref[...] = 2.0 * v
    dt2_ref[...] = jnp.log(dts_ref[...] + 1e-08)


NC, NS, L = 2, 16, 16
NW = NC * NS
PER_W = N // NW
CH = 512
NCHUNK = PER_W // CH
IDX_SUB = 128

_sc_mesh = plsc.VectorSubcoreMesh(core_axis_name="c", subcore_axis_name="s")


@functools.partial(
    pl.kernel,
    out_type=jax.ShapeDtypeStruct((N, EMBED), jnp.float32),
    mesh=_sc_mesh,
    scratch_types=[
        pltpu.VMEM((CH // IDX_SUB, 1, IDX_SUB), jnp.int32),
        pltpu.VMEM((CH,), jnp.float32),
        pltpu.VMEM((CH, EMBED), jnp.float32),
        pltpu.VMEM((1, EMBED), jnp.float32),
        pltpu.SemaphoreType.DMA,
    ],
    compiler_params=pltpu.CompilerParams(
        needs_layout_passes=False, use_tc_tiling_on_sc=False),
)
def _sc_main(dt2_hbm, types_hbm, vrow_hbm, dtab_hbm, out_hbm,
             idx_v, dt_v, rows_v, v_v, sem):
    wid = lax.axis_index("s") * NC + lax.axis_index("c")
    pltpu.sync_copy(vrow_hbm, v_v)
    v2 = [v_v[0, pl.ds(q * L, L)] for q in range(EMBED // L)]
    zeros16 = jnp.zeros((L,), jnp.int32)
    base = wid * PER_W

    def chunk_body(ci, _):
        el0 = base + ci * CH
        pltpu.sync_copy(types_hbm.at[pl.ds(el0 // IDX_SUB, CH // IDX_SUB)], idx_v)
        pltpu.sync_copy(dt2_hbm.at[pl.ds(el0, CH)], dt_v)
        for t in range(CH // IDX_SUB):
            pltpu.async_copy(dtab_hbm.at[idx_v.at[t, 0]],
                             rows_v.at[pl.ds(t * IDX_SUB, IDX_SUB)], sem).wait()

        @plsc.parallel_loop(0, CH, 1, unroll=4)
        def _elem(e):
            dts = plsc.load_gather(dt_v, [zeros16 + e])
            for q in range(EMBED // L):
                g2 = rows_v[e, pl.ds(q * L, L)]
                z2 = jnp.clip(g2 + dts * v2[q], -60.0, 60.0)
                u = jnp.exp(z2)
                rows_v[e, pl.ds(q * L, L)] = (u - 1.0) / (u + 1.0)

        pltpu.sync_copy(rows_v, out_hbm.at[pl.ds(el0, CH)])
        return 0

    lax.fori_loop(0, NCHUNK, chunk_body, 0)


def kernel(seq_dts, seq_types, type_table, time_W, time_b, enc_W, enc_b):
    dts2 = seq_dts.reshape(N // 128, 128)
    table_pad = jnp.pad(type_table, ((0, TPAD - (NTYPES + 1)), (0, 0)))
    tb2 = time_b.reshape(1, HALF)
    eb2 = enc_b.reshape(1, EMBED)

    dt2, dtab, vrow = pl.pallas_call(
        _tc_prologue,
        grid=(PRO_GRID,),
        in_specs=[
            pl.BlockSpec((PRO_ROWS, 128), lambda i: (i, 0)),
            pl.BlockSpec((TPAD, HALF), lambda i: (0, 0)),
            pl.BlockSpec((1, HALF), lambda i: (0, 0)),
            pl.BlockSpec((1, HALF), lambda i: (0, 0)),
            pl.BlockSpec((EMBED, EMBED), lambda i: (0, 0)),
            pl.BlockSpec((1, EMBED), lambda i: (0, 0)),
        ],
        out_specs=[
            pl.BlockSpec((PRO_ROWS, 128), lambda i: (i, 0)),
            pl.BlockSpec((TPAD, EMBED), lambda i: (0, 0)),
            pl.BlockSpec((1, EMBED), lambda i: (0, 0)),
        ],
        out_shape=[
            jax.ShapeDtypeStruct((N // 128, 128), jnp.float32),
            jax.ShapeDtypeStruct((TPAD, EMBED), jnp.float32),
            jax.ShapeDtypeStruct((1, EMBED), jnp.float32),
        ],
    )(dts2, table_pad, time_W, tb2, enc_W, eb2)

    types3 = seq_types.astype(jnp.int32).reshape(N // IDX_SUB, 1, IDX_SUB)
    out = _sc_main(dt2.reshape(N), types3, vrow, dtab)
    return out.reshape(B, S, EMBED)


# speedup vs baseline: 1.0160x; 1.0160x over previous
import functools
import math

import jax
import jax.numpy as jnp
from jax import lax
from jax.experimental import pallas as pl
from jax.experimental.pallas import tpu as pltpu
from jax.experimental.pallas import tpu_sc as plsc

EMBED = 64
HALF = 32
NTYPES = 100
TPAD = 128

B, S = 4096, 200
N = B * S


PRO_ROWS = 128
PRO_GRID = (N // 128) // PRO_ROWS


def _tc_prologue(dts_ref, table_ref, tw_ref, tb_ref, ew_ref, eb_ref,
                 dt2_ref, dtab_ref, vrow_ref):
    ftab = (table_ref[...] * math.sqrt(EMBED // 2)) @ ew_ref[HALF:, :]
    c = tb_ref[...] @ ew_ref[:HALF, :] + eb_ref[...]
    v = tw_ref[...] @ ew_ref[:HALF, :]
    dtab_ref[...] = 2.0 * (ftab + c)
    vrow_ref[...] = 2.0 * v
    dt2_ref[...] = jnp.log(dts_ref[...] + 1e-08)


NC, NS, L = 2, 16, 16
NW = NC * NS
PER_W = N // NW
CH = 512
NCHUNK = PER_W // CH
IDX_SUB = 128

_sc_mesh = plsc.VectorSubcoreMesh(core_axis_name="c", subcore_axis_name="s")


@functools.partial(
    pl.kernel,
    out_type=jax.ShapeDtypeStruct((N, EMBED), jnp.float32),
    mesh=_sc_mesh,
    scratch_types=[
        pltpu.VMEM((CH // IDX_SUB, 1, IDX_SUB), jnp.int32),
        pltpu.VMEM((CH,), jnp.float32),
        pltpu.VMEM((CH, EMBED), jnp.float32),
        pltpu.VMEM((1, EMBED), jnp.float32),
        pltpu.SemaphoreType.DMA,
    ],
    compiler_params=pltpu.CompilerParams(
        needs_layout_passes=False, use_tc_tiling_on_sc=False),
)
def _sc_main(dt2_hbm, types_hbm, vrow_hbm, dtab_hbm, out_hbm,
             idx_v, dt_v, rows_v, v_v, sem):
    wid = lax.axis_index("s") * NC + lax.axis_index("c")
    pltpu.sync_copy(vrow_hbm, v_v)
    v2 = [v_v[0, pl.ds(q * L, L)] for q in range(EMBED // L)]
    zeros16 = jnp.zeros((L,), jnp.int32)
    base = wid * PER_W

    def chunk_body(ci, _):
        el0 = base + ci * CH
        pltpu.sync_copy(types_hbm.at[pl.ds(el0 // IDX_SUB, CH // IDX_SUB)], idx_v)
        pltpu.sync_copy(dt2_hbm.at[pl.ds(el0, CH)], dt_v)
        copies = [
            pltpu.async_copy(dtab_hbm.at[idx_v.at[t, 0]],
                             rows_v.at[pl.ds(t * IDX_SUB, IDX_SUB)], sem)
            for t in range(CH // IDX_SUB)
        ]
        for cp in copies:
            cp.wait()

        @plsc.parallel_loop(0, CH, 1, unroll=8)
        def _elem(e):
            dts = plsc.load_gather(dt_v, [zeros16 + e])
            for q in range(EMBED // L):
                g2 = rows_v[e, pl.ds(q * L, L)]
                u = jnp.exp(jnp.minimum(g2 + dts * v2[q], 60.0))
                rows_v[e, pl.ds(q * L, L)] = (u - 1.0) / (u + 1.0)

        pltpu.sync_copy(rows_v, out_hbm.at[pl.ds(el0, CH)])
        return 0

    lax.fori_loop(0, NCHUNK, chunk_body, 0)


def kernel(seq_dts, seq_types, type_table, time_W, time_b, enc_W, enc_b):
    dts2 = seq_dts.reshape(N // 128, 128)
    table_pad = jnp.pad(type_table, ((0, TPAD - (NTYPES + 1)), (0, 0)))
    tb2 = time_b.reshape(1, HALF)
    eb2 = enc_b.reshape(1, EMBED)

    dt2, dtab, vrow = pl.pallas_call(
        _tc_prologue,
        grid=(PRO_GRID,),
        in_specs=[
            pl.BlockSpec((PRO_ROWS, 128), lambda i: (i, 0)),
            pl.BlockSpec((TPAD, HALF), lambda i: (0, 0)),
            pl.BlockSpec((1, HALF), lambda i: (0, 0)),
            pl.BlockSpec((1, HALF), lambda i: (0, 0)),
            pl.BlockSpec((EMBED, EMBED), lambda i: (0, 0)),
            pl.BlockSpec((1, EMBED), lambda i: (0, 0)),
        ],
        out_specs=[
            pl.BlockSpec((PRO_ROWS, 128), lambda i: (i, 0)),
            pl.BlockSpec((TPAD, EMBED), lambda i: (0, 0)),
            pl.BlockSpec((1, EMBED), lambda i: (0, 0)),
        ],
        out_shape=[
            jax.ShapeDtypeStruct((N // 128, 128), jnp.float32),
            jax.ShapeDtypeStruct((TPAD, EMBED), jnp.float32),
            jax.ShapeDtypeStruct((1, EMBED), jnp.float32),
        ],
    )(dts2, table_pad, time_W, tb2, enc_W, eb2)

    types3 = seq_types.astype(jnp.int32).reshape(N // IDX_SUB, 1, IDX_SUB)
    out = _sc_main(dt2.reshape(N), types3, vrow, dtab)
    return out.reshape(B, S, EMBED)


# speedup vs baseline: 1.3965x; 1.3745x over previous
import functools
import math

import jax
import jax.numpy as jnp
from jax import lax
from jax.experimental import pallas as pl
from jax.experimental.pallas import tpu as pltpu
from jax.experimental.pallas import tpu_sc as plsc

EMBED = 64
HALF = 32
NTYPES = 100
TPAD = 128

B, S = 4096, 200
N = B * S

SC_N = 204800
TC_N = N - SC_N


PRO_ROWS = SC_N // 128


def _tc_prologue(dts_ref, table_ref, tw_ref, tb_ref, ew_ref, eb_ref,
                 dt2_ref, dtab_ref, vrow_ref):
    ftab = (table_ref[...] * math.sqrt(EMBED // 2)) @ ew_ref[HALF:, :]
    c = tb_ref[...] @ ew_ref[:HALF, :] + eb_ref[...]
    v = tw_ref[...] @ ew_ref[:HALF, :]
    dtab_ref[...] = 2.0 * (ftab + c)
    vrow_ref[...] = 2.0 * v
    dt2_ref[...] = jnp.log(dts_ref[...] + 1e-08)


NC, NS, L = 2, 16, 16
NW = NC * NS
PER_W = SC_N // NW
CH = 256
NCHUNK = PER_W // CH
IDX_SUB = 128

_sc_mesh = plsc.VectorSubcoreMesh(core_axis_name="c", subcore_axis_name="s")


@functools.partial(
    pl.kernel,
    out_type=jax.ShapeDtypeStruct((SC_N, EMBED), jnp.float32),
    mesh=_sc_mesh,
    scratch_types=[
        pltpu.VMEM((CH // IDX_SUB, 1, IDX_SUB), jnp.int32),
        pltpu.VMEM((CH,), jnp.float32),
        pltpu.VMEM((CH, EMBED), jnp.float32),
        pltpu.VMEM((1, EMBED), jnp.float32),
        pltpu.SemaphoreType.DMA,
    ],
    compiler_params=pltpu.CompilerParams(
        needs_layout_passes=False, use_tc_tiling_on_sc=False),
)
def _sc_main(dt2_hbm, types_hbm, vrow_hbm, dtab_hbm, out_hbm,
             idx_v, dt_v, rows_v, v_v, sem):
    wid = lax.axis_index("s") * NC + lax.axis_index("c")
    pltpu.sync_copy(vrow_hbm, v_v)
    v2 = [v_v[0, pl.ds(q * L, L)] for q in range(EMBED // L)]
    zeros16 = jnp.zeros((L,), jnp.int32)
    base = wid * PER_W

    def chunk_body(ci, _):
        el0 = base + ci * CH
        pltpu.sync_copy(types_hbm.at[pl.ds(el0 // IDX_SUB, CH // IDX_SUB)], idx_v)
        pltpu.sync_copy(dt2_hbm.at[pl.ds(el0, CH)], dt_v)
        copies = [
            pltpu.async_copy(dtab_hbm.at[idx_v.at[t, 0]],
                             rows_v.at[pl.ds(t * IDX_SUB, IDX_SUB)], sem)
            for t in range(CH // IDX_SUB)
        ]
        for cp in copies:
            cp.wait()

        @plsc.parallel_loop(0, CH, 1, unroll=8)
        def _elem(e):
            dts = plsc.load_gather(dt_v, [zeros16 + e])
            for q in range(EMBED // L):
                g2 = rows_v[e, pl.ds(q * L, L)]
                u = jnp.exp(jnp.minimum(g2 + dts * v2[q], 60.0))
                rows_v[e, pl.ds(q * L, L)] = (u - 1.0) / (u + 1.0)

        pltpu.sync_copy(rows_v, out_hbm.at[pl.ds(el0, CH)])
        return 0

    lax.fori_loop(0, NCHUNK, chunk_body, 0)


BLK = 8192
TC_GRID = TC_N // BLK


def _tc_body(dts_ref, types_ref, table_ref, tw_ref, tb_ref, ew_ref, eb_ref,
             out_ref):
    ftab = (table_ref[...] * math.sqrt(EMBED // 2)) @ ew_ref[HALF:, :]
    c = tb_ref[...] @ ew_ref[:HALF, :] + eb_ref[...]
    v = tw_ref[...] @ ew_ref[:HALF, :]
    row = jax.lax.broadcasted_iota(jnp.int32, (TPAD, EMBED), 0)
    ftab_full = jnp.where(row == TPAD - 1, v, ftab + c)

    types = types_ref[0]
    dt = jnp.log(dts_ref[0] + 1e-08)
    tid = jax.lax.broadcasted_iota(jnp.int32, (TPAD, BLK), 0)
    onehot_t = (tid == types).astype(jnp.float32)
    lhs = jnp.where(tid == TPAD - 1, dt, onehot_t)

    z = jax.lax.dot_general(lhs, ftab_full,
                            dimension_numbers=(((0,), (0,)), ((), ())),
                            preferred_element_type=jnp.float32)
    out_ref[...] = jnp.tanh(z)


def kernel(seq_dts, seq_types, type_table, time_W, time_b, enc_W, enc_b):
    dts_flat = seq_dts.reshape(N)
    types_flat = seq_types.astype(jnp.int32).reshape(N)
    table_pad = jnp.pad(type_table, ((0, TPAD - (NTYPES + 1)), (0, 0)))
    tw2 = time_W.reshape(1, HALF)
    tb2 = time_b.reshape(1, HALF)
    eb2 = enc_b.reshape(1, EMBED)

    dts_sc = dts_flat[:SC_N].reshape(PRO_ROWS, 128)
    dt2, dtab, vrow = pl.pallas_call(
        _tc_prologue,
        grid=(1,),
        in_specs=[
            pl.BlockSpec((PRO_ROWS, 128), lambda i: (0, 0)),
            pl.BlockSpec((TPAD, HALF), lambda i: (0, 0)),
            pl.BlockSpec((1, HALF), lambda i: (0, 0)),
            pl.BlockSpec((1, HALF), lambda i: (0, 0)),
            pl.BlockSpec((EMBED, EMBED), lambda i: (0, 0)),
            pl.BlockSpec((1, EMBED), lambda i: (0, 0)),
        ],
        out_specs=[
            pl.BlockSpec((PRO_ROWS, 128), lambda i: (0, 0)),
            pl.BlockSpec((TPAD, EMBED), lambda i: (0, 0)),
            pl.BlockSpec((1, EMBED), lambda i: (0, 0)),
        ],
        out_shape=[
            jax.ShapeDtypeStruct((PRO_ROWS, 128), jnp.float32),
            jax.ShapeDtypeStruct((TPAD, EMBED), jnp.float32),
            jax.ShapeDtypeStruct((1, EMBED), jnp.float32),
        ],
    )(dts_sc, table_pad, tw2, tb2, enc_W, eb2)

    types_sc = types_flat[:SC_N].reshape(SC_N // IDX_SUB, 1, IDX_SUB)
    sc_out = _sc_main(dt2.reshape(SC_N), types_sc, vrow, dtab)

    dts_tc = dts_flat[SC_N:].reshape(TC_GRID, 1, BLK)
    types_tc = types_flat[SC_N:].reshape(TC_GRID, 1, BLK)
    tc_out = pl.pallas_call(
        _tc_body,
        grid=(TC_GRID,),
        in_specs=[
            pl.BlockSpec((1, 1, BLK), lambda i: (i, 0, 0)),
            pl.BlockSpec((1, 1, BLK), lambda i: (i, 0, 0)),
            pl.BlockSpec((TPAD, HALF), lambda i: (0, 0)),
            pl.BlockSpec((1, HALF), lambda i: (0, 0)),
            pl.BlockSpec((1, HALF), lambda i: (0, 0)),
            pl.BlockSpec((EMBED, EMBED), lambda i: (0, 0)),
            pl.BlockSpec((1, EMBED), lambda i: (0, 0)),
        ],
        out_specs=pl.BlockSpec((BLK, EMBED), lambda i: (i, 0)),
        out_shape=jax.ShapeDtypeStruct((TC_N, EMBED), jnp.float32),
    )(dts_tc, types_tc, table_pad, tw2, tb2, enc_W, eb2)

    out = jnp.concatenate([sc_out, tc_out], axis=0)
    return out.reshape(B, S, EMBED)
